# Initial kernel scaffold; baseline (speedup 1.0000x reference)
import jax
import jax.numpy as jnp
from jax.experimental import pallas as pl


def kernel(x, edge_index, W1, b1, W2, b2, Wd1, bd1, Wd2, bd2):
    raise NotImplementedError("write your pallas kernel here")


# speedup vs baseline: 5.4426x; 5.4426x over previous
import jax
import jax.numpy as jnp
from jax import lax
from jax.experimental import pallas as pl
from jax.experimental.pallas import tpu as pltpu
from jax.experimental.pallas import tpu_sc as plsc

N = 10000
E = 320000
D = 128
NC = 2
NS = 16
NW = NC * NS
EPW = E // NW
C = 80
NCHUNK = EPW // C
RPT = 624
TAIL_BASE = NS * RPT
TAIL = N - TAIL_BASE
NZC = 7

_MESH = plsc.VectorSubcoreMesh(
    core_axis_name="c", subcore_axis_name="s", num_cores=NC, num_subcores=NS)


def _fill2d(ref, nrows, ncols, val):
    nk = ncols // 16

    def body(i, _):
        r = i // nk
        k = i % nk
        ref[r, pl.ds(k * 16, 16)] = jnp.full((16,), val, jnp.float32)
        return 0

    lax.fori_loop(0, nrows * nk, body, 0)


def _fill1d(ref, n, val):
    def body(i, _):
        ref[pl.ds(i * 16, 16)] = jnp.full((16,), val, jnp.float32)
        return 0

    lax.fori_loop(0, n // 16, body, 0)


def _zero_shared2d(zbuf, shared, base, s):
    for j in range(NZC):
        pltpu.sync_copy(zbuf, shared.at[pl.ds(base + j * C, C), :])
    pltpu.sync_copy(zbuf.at[pl.ds(0, RPT - NZC * C), :],
                    shared.at[pl.ds(base + NZC * C, RPT - NZC * C), :])

    @pl.when(s == NS - 1)
    def _():
        pltpu.sync_copy(zbuf.at[pl.ds(0, TAIL), :],
                        shared.at[pl.ds(TAIL_BASE, TAIL), :])


def _writeback2d(shared, out_hbm, c, base, s):
    pltpu.sync_copy(shared.at[pl.ds(base, RPT), :],
                    out_hbm.at[c, pl.ds(base, RPT), :])

    @pl.when(s == NS - 1)
    def _():
        pltpu.sync_copy(shared.at[pl.ds(TAIL_BASE, TAIL), :],
                        out_hbm.at[c, pl.ds(TAIL_BASE, TAIL), :])


def _sc_agg_deg_body(x_hbm, src_hbm, dst_hbm, agg_hbm, deg_hbm,
                     sidx, didx, rows, ones, degbuf, acc_sh, deg_sh, sem):
    c = lax.axis_index("c")
    s = lax.axis_index("s")
    wid = s * NC + c
    base = s * RPT

    _fill2d(rows, C, D, 0.0)
    _fill1d(ones, C, 0.0)
    _zero_shared2d(rows, acc_sh, base, s)
    for j in range(NZC):
        pltpu.sync_copy(ones, deg_sh.at[pl.ds(base + j * C, C)])
    pltpu.sync_copy(ones.at[pl.ds(0, RPT - NZC * C)],
                    deg_sh.at[pl.ds(base + NZC * C, RPT - NZC * C)])

    @pl.when(s == NS - 1)
    def _():
        pltpu.sync_copy(ones.at[pl.ds(0, TAIL)],
                        deg_sh.at[pl.ds(TAIL_BASE, TAIL)])

    _fill1d(ones, C, 1.0)
    plsc.subcore_barrier()

    ebase = wid * EPW

    def body(i, _):
        off = ebase + i * C
        pltpu.sync_copy(src_hbm.at[pl.ds(off, C)], sidx)
        pltpu.sync_copy(dst_hbm.at[pl.ds(off, C)], didx)
        pltpu.async_copy(x_hbm.at[sidx], rows, sem).wait()
        pltpu.sync_copy(rows, acc_sh.at[didx], add=True)
        pltpu.sync_copy(ones, deg_sh.at[didx], add=True)
        return 0

    lax.fori_loop(0, NCHUNK, body, 0)
    plsc.subcore_barrier()

    _writeback2d(acc_sh, agg_hbm, c, base, s)
    pltpu.sync_copy(deg_sh.at[pl.ds(base, RPT)], degbuf)
    pltpu.sync_copy(degbuf, deg_hbm.at[pl.ds(c * N + base, RPT)])

    @pl.when(s == NS - 1)
    def _():
        pltpu.sync_copy(deg_sh.at[pl.ds(TAIL_BASE, TAIL)],
                        degbuf.at[pl.ds(0, TAIL)])
        pltpu.sync_copy(degbuf.at[pl.ds(0, TAIL)],
                        deg_hbm.at[pl.ds(c * N + TAIL_BASE, TAIL)])


_sc_agg_deg = pl.kernel(
    _sc_agg_deg_body,
    out_type=(jax.ShapeDtypeStruct((NC, N, D), jnp.float32),
              jax.ShapeDtypeStruct((NC * N,), jnp.float32)),
    mesh=_MESH,
    scratch_types=[
        pltpu.VMEM((C,), jnp.int32),
        pltpu.VMEM((C,), jnp.int32),
        pltpu.VMEM((C, D), jnp.float32),
        pltpu.VMEM((C,), jnp.float32),
        pltpu.VMEM((RPT,), jnp.float32),
        pltpu.VMEM_SHARED((N, D), jnp.float32),
        pltpu.VMEM_SHARED((N,), jnp.float32),
        pltpu.SemaphoreType.DMA,
    ],
)


def _sc_agg_body(h_hbm, src_hbm, dst_hbm, agg_hbm,
                 sidx, didx, rows, acc_sh, sem):
    c = lax.axis_index("c")
    s = lax.axis_index("s")
    wid = s * NC + c
    base = s * RPT

    _fill2d(rows, C, D, 0.0)
    _zero_shared2d(rows, acc_sh, base, s)
    plsc.subcore_barrier()

    ebase = wid * EPW

    def body(i, _):
        off = ebase + i * C
        pltpu.sync_copy(src_hbm.at[pl.ds(off, C)], sidx)
        pltpu.sync_copy(dst_hbm.at[pl.ds(off, C)], didx)
        pltpu.async_copy(h_hbm.at[sidx], rows, sem).wait()
        pltpu.sync_copy(rows, acc_sh.at[didx], add=True)
        return 0

    lax.fori_loop(0, NCHUNK, body, 0)
    plsc.subcore_barrier()

    _writeback2d(acc_sh, agg_hbm, c, base, s)


_sc_agg = pl.kernel(
    _sc_agg_body,
    out_type=jax.ShapeDtypeStruct((NC, N, D), jnp.float32),
    mesh=_MESH,
    scratch_types=[
        pltpu.VMEM((C,), jnp.int32),
        pltpu.VMEM((C,), jnp.int32),
        pltpu.VMEM((C, D), jnp.float32),
        pltpu.VMEM_SHARED((N, D), jnp.float32),
        pltpu.SemaphoreType.DMA,
    ],
)

BN = 1000


def _tc1_body(p_ref, x_ref, dp_ref, w_ref, b_ref, o_ref):
    inv = 1.0 / (dp_ref[0, :, 0:1] + dp_ref[1, :, 0:1] + 1.0)
    hn = (p_ref[0] + p_ref[1] + x_ref[...]) * inv
    z = jnp.dot(hn, w_ref[...], preferred_element_type=jnp.float32) + b_ref[...]
    o_ref[...] = jnp.maximum(z, 0.0)


def _tc1(p, x, dp, w1, b1):
    return pl.pallas_call(
        _tc1_body,
        grid=(N // BN,),
        in_specs=[
            pl.BlockSpec((NC, BN, D), lambda i: (0, i, 0)),
            pl.BlockSpec((BN, D), lambda i: (i, 0)),
            pl.BlockSpec((NC, BN, 1), lambda i: (0, i, 0)),
            pl.BlockSpec((D, D), lambda i: (0, 0)),
            pl.BlockSpec((1, D), lambda i: (0, 0)),
        ],
        out_specs=pl.BlockSpec((BN, D), lambda i: (i, 0)),
        out_shape=jax.ShapeDtypeStruct((N, D), jnp.float32),
    )(p, x, dp, w1, b1)


def _tc2_body(q_ref, h1_ref, dp_ref, w2_ref, b2_ref, wd1_ref, bd1_ref,
              wd2_ref, bd2_ref, o_ref):
    inv = 1.0 / (dp_ref[0, :, 0:1] + dp_ref[1, :, 0:1] + 1.0)
    hn = (q_ref[0] + q_ref[1] + h1_ref[...]) * inv
    h2 = jnp.dot(hn, w2_ref[...], preferred_element_type=jnp.float32) + b2_ref[...]
    t = jnp.maximum(
        jnp.dot(h2, wd1_ref[...], preferred_element_type=jnp.float32)
        + bd1_ref[...], 0.0)
    o_ref[...] = (jnp.dot(t, wd2_ref[...], preferred_element_type=jnp.float32)
                  + bd2_ref[...])


def _tc2(q, h1, dp, w2, b2, wd1, bd1, wd2, bd2):
    return pl.pallas_call(
        _tc2_body,
        grid=(N // BN,),
        in_specs=[
            pl.BlockSpec((NC, BN, D), lambda i: (0, i, 0)),
            pl.BlockSpec((BN, D), lambda i: (i, 0)),
            pl.BlockSpec((NC, BN, 1), lambda i: (0, i, 0)),
            pl.BlockSpec((D, D), lambda i: (0, 0)),
            pl.BlockSpec((1, D), lambda i: (0, 0)),
            pl.BlockSpec((D, D), lambda i: (0, 0)),
            pl.BlockSpec((1, D), lambda i: (0, 0)),
            pl.BlockSpec((D, 1), lambda i: (0, 0)),
            pl.BlockSpec((1, 1), lambda i: (0, 0)),
        ],
        out_specs=pl.BlockSpec((BN, 1), lambda i: (i, 0)),
        out_shape=jax.ShapeDtypeStruct((N, 1), jnp.float32),
    )(q, h1, dp, w2, b2, wd1, bd1, wd2, bd2)


def kernel(x, edge_index, W1, b1, W2, b2, Wd1, bd1, Wd2, bd2):
    src = edge_index[0]
    dst = edge_index[1]
    agg_p, deg_flat = _sc_agg_deg(x, src, dst)
    deg_p = deg_flat.reshape(NC, N, 1)
    h1 = _tc1(agg_p, x, deg_p, W1, b1.reshape(1, D))
    agg2_p = _sc_agg(h1, src, dst)
    out = _tc2(agg2_p, h1, deg_p, W2, b2.reshape(1, D),
               Wd1, bd1.reshape(1, D), Wd2, bd2.reshape(1, 1))
    return out


# speedup vs baseline: 12.6379x; 2.3220x over previous
import jax
import jax.numpy as jnp
from jax import lax
from jax.experimental import pallas as pl
from jax.experimental.pallas import tpu as pltpu
from jax.experimental.pallas import tpu_sc as plsc

N = 10000
E = 320000
D = 128
NC = 2
NS = 16
NW = NC * NS
EPW = E // NW
C = 40
NCHUNK = EPW // C
NBUF = 5
NROUNDS = NCHUNK // NBUF
RPT = 624
TAIL_BASE = NS * RPT
TAIL = N - TAIL_BASE
NZC = 15

_MESH = plsc.VectorSubcoreMesh(
    core_axis_name="c", subcore_axis_name="s", num_cores=NC, num_subcores=NS)


def _fill2d(ref, nrows, ncols, val):
    nk = ncols // 16

    def body(i, _):
        r = i // nk
        k = i % nk
        ref[r, pl.ds(k * 16, 16)] = jnp.full((16,), val, jnp.float32)
        return 0

    lax.fori_loop(0, nrows * nk, body, 0)


def _fill1d_40(ref, val):
    for off in (0, 16, 24):
        ref[pl.ds(off, 16)] = jnp.full((16,), val, jnp.float32)


def _zero_shared2d(zbuf, shared, base, s):
    for j in range(NZC):
        pltpu.sync_copy(zbuf, shared.at[pl.ds(base + j * C, C), :])
    pltpu.sync_copy(zbuf.at[pl.ds(0, RPT - NZC * C), :],
                    shared.at[pl.ds(base + NZC * C, RPT - NZC * C), :])

    @pl.when(s == NS - 1)
    def _():
        pltpu.sync_copy(zbuf.at[pl.ds(0, TAIL), :],
                        shared.at[pl.ds(TAIL_BASE, TAIL), :])


def _writeback2d(shared, out_hbm, c, base, s):
    pltpu.sync_copy(shared.at[pl.ds(base, RPT), :],
                    out_hbm.at[c, pl.ds(base, RPT), :])

    @pl.when(s == NS - 1)
    def _():
        pltpu.sync_copy(shared.at[pl.ds(TAIL_BASE, TAIL), :],
                        out_hbm.at[c, pl.ds(TAIL_BASE, TAIL), :])


def _sc_agg_deg_body(x_hbm, src_hbm, dst_hbm, agg_hbm, deg_hbm,
                     sidx, didx0, didx1, didx2, didx3, didx4, ones, degbuf,
                     rows0, rows1, rows2, rows3, rows4,
                     acc_sh, deg_sh, gsem, ssem, isem, dsem):
    rows = (rows0, rows1, rows2, rows3, rows4)
    didx = (didx0, didx1, didx2, didx3, didx4)
    c = lax.axis_index("c")
    s = lax.axis_index("s")
    wid = s * NC + c
    base = s * RPT
    ebase = wid * EPW

    _fill2d(rows0, C, D, 0.0)
    _fill1d_40(ones, 0.0)
    _zero_shared2d(rows0, acc_sh, base, s)
    for j in range(NZC):
        pltpu.sync_copy(ones, deg_sh.at[pl.ds(base + j * C, C)])
    pltpu.sync_copy(ones.at[pl.ds(0, RPT - NZC * C)],
                    deg_sh.at[pl.ds(base + NZC * C, RPT - NZC * C)])

    @pl.when(s == NS - 1)
    def _():
        pltpu.sync_copy(ones.at[pl.ds(0, TAIL)],
                        deg_sh.at[pl.ds(TAIL_BASE, TAIL)])

    _fill1d_40(ones, 1.0)
    pltpu.sync_copy(src_hbm.at[pl.ds(ebase, EPW)], sidx)
    plsc.subcore_barrier()

    def round_(j, _):
        for b in range(NBUF):
            i = j * NBUF + b

            @pl.when(j > 0)
            def _():
                pltpu.make_async_copy(
                    rows[b], acc_sh.at[didx[b]], ssem.at[b]).wait()
                pltpu.make_async_copy(
                    ones, deg_sh.at[didx[b]], dsem.at[b]).wait()

            pltpu.async_copy(dst_hbm.at[pl.ds(ebase + i * C, C)],
                             didx[b], isem.at[b])
            pltpu.async_copy(x_hbm.at[sidx.at[pl.ds(i * C, C)]],
                             rows[b], gsem.at[b])
        for b in range(NBUF):
            i = j * NBUF + b
            pltpu.make_async_copy(
                dst_hbm.at[pl.ds(ebase + i * C, C)], didx[b],
                isem.at[b]).wait()
            pltpu.make_async_copy(
                x_hbm.at[sidx.at[pl.ds(i * C, C)]], rows[b],
                gsem.at[b]).wait()
            pltpu.async_copy(rows[b], acc_sh.at[didx[b]], ssem.at[b],
                             add=True)
            pltpu.async_copy(ones, deg_sh.at[didx[b]], dsem.at[b],
                             add=True)
        return 0

    lax.fori_loop(0, NROUNDS, round_, 0)
    for b in range(NBUF):
        pltpu.make_async_copy(rows[b], acc_sh.at[didx[b]],
                              ssem.at[b]).wait()
        pltpu.make_async_copy(ones, deg_sh.at[didx[b]],
                              dsem.at[b]).wait()
    plsc.subcore_barrier()

    _writeback2d(acc_sh, agg_hbm, c, base, s)
    pltpu.sync_copy(deg_sh.at[pl.ds(base, RPT)], degbuf)
    pltpu.sync_copy(degbuf, deg_hbm.at[pl.ds(c * N + base, RPT)])

    @pl.when(s == NS - 1)
    def _():
        pltpu.sync_copy(deg_sh.at[pl.ds(TAIL_BASE, TAIL)],
                        degbuf.at[pl.ds(0, TAIL)])
        pltpu.sync_copy(degbuf.at[pl.ds(0, TAIL)],
                        deg_hbm.at[pl.ds(c * N + TAIL_BASE, TAIL)])


_sc_agg_deg = pl.kernel(
    _sc_agg_deg_body,
    out_type=(jax.ShapeDtypeStruct((NC, N, D), jnp.float32),
              jax.ShapeDtypeStruct((NC * N,), jnp.float32)),
    mesh=_MESH,
    scratch_types=[
        pltpu.VMEM((EPW,), jnp.int32),
        pltpu.VMEM((C,), jnp.int32),
        pltpu.VMEM((C,), jnp.int32),
        pltpu.VMEM((C,), jnp.int32),
        pltpu.VMEM((C,), jnp.int32),
        pltpu.VMEM((C,), jnp.int32),
        pltpu.VMEM((C,), jnp.float32),
        pltpu.VMEM((RPT,), jnp.float32),
        pltpu.VMEM((C, D), jnp.float32),
        pltpu.VMEM((C, D), jnp.float32),
        pltpu.VMEM((C, D), jnp.float32),
        pltpu.VMEM((C, D), jnp.float32),
        pltpu.VMEM((C, D), jnp.float32),
        pltpu.VMEM_SHARED((N, D), jnp.float32),
        pltpu.VMEM_SHARED((N,), jnp.float32),
        pltpu.SemaphoreType.DMA((NBUF,)),
        pltpu.SemaphoreType.DMA((NBUF,)),
        pltpu.SemaphoreType.DMA((NBUF,)),
        pltpu.SemaphoreType.DMA((NBUF,)),
    ],
)


def _sc_agg_body(h_hbm, src_hbm, dst_hbm, agg_hbm,
                 sidx, didx0, didx1, didx2, didx3, didx4,
                 rows0, rows1, rows2, rows3, rows4,
                 acc_sh, gsem, ssem, isem):
    rows = (rows0, rows1, rows2, rows3, rows4)
    didx = (didx0, didx1, didx2, didx3, didx4)
    c = lax.axis_index("c")
    s = lax.axis_index("s")
    wid = s * NC + c
    base = s * RPT
    ebase = wid * EPW

    _fill2d(rows0, C, D, 0.0)
    _zero_shared2d(rows0, acc_sh, base, s)
    pltpu.sync_copy(src_hbm.at[pl.ds(ebase, EPW)], sidx)
    plsc.subcore_barrier()

    def round_(j, _):
        for b in range(NBUF):
            i = j * NBUF + b

            @pl.when(j > 0)
            def _():
                pltpu.make_async_copy(
                    rows[b], acc_sh.at[didx[b]], ssem.at[b]).wait()

            pltpu.async_copy(dst_hbm.at[pl.ds(ebase + i * C, C)],
                             didx[b], isem.at[b])
            pltpu.async_copy(h_hbm.at[sidx.at[pl.ds(i * C, C)]],
                             rows[b], gsem.at[b])
        for b in range(NBUF):
            i = j * NBUF + b
            pltpu.make_async_copy(
                dst_hbm.at[pl.ds(ebase + i * C, C)], didx[b],
                isem.at[b]).wait()
            pltpu.make_async_copy(
                h_hbm.at[sidx.at[pl.ds(i * C, C)]], rows[b],
                gsem.at[b]).wait()
            pltpu.async_copy(rows[b], acc_sh.at[didx[b]], ssem.at[b],
                             add=True)
        return 0

    lax.fori_loop(0, NROUNDS, round_, 0)
    for b in range(NBUF):
        pltpu.make_async_copy(rows[b], acc_sh.at[didx[b]],
                              ssem.at[b]).wait()
    plsc.subcore_barrier()

    _writeback2d(acc_sh, agg_hbm, c, base, s)


_sc_agg = pl.kernel(
    _sc_agg_body,
    out_type=jax.ShapeDtypeStruct((NC, N, D), jnp.float32),
    mesh=_MESH,
    scratch_types=[
        pltpu.VMEM((EPW,), jnp.int32),
        pltpu.VMEM((C,), jnp.int32),
        pltpu.VMEM((C,), jnp.int32),
        pltpu.VMEM((C,), jnp.int32),
        pltpu.VMEM((C,), jnp.int32),
        pltpu.VMEM((C,), jnp.int32),
        pltpu.VMEM((C, D), jnp.float32),
        pltpu.VMEM((C, D), jnp.float32),
        pltpu.VMEM((C, D), jnp.float32),
        pltpu.VMEM((C, D), jnp.float32),
        pltpu.VMEM((C, D), jnp.float32),
        pltpu.VMEM_SHARED((N, D), jnp.float32),
        pltpu.SemaphoreType.DMA((NBUF,)),
        pltpu.SemaphoreType.DMA((NBUF,)),
        pltpu.SemaphoreType.DMA((NBUF,)),
    ],
)

BN = 1000


def _tc1_body(p_ref, x_ref, dp_ref, w_ref, b_ref, o_ref):
    inv = 1.0 / (dp_ref[0, :, 0:1] + dp_ref[1, :, 0:1] + 1.0)
    hn = (p_ref[0] + p_ref[1] + x_ref[...]) * inv
    z = jnp.dot(hn, w_ref[...], preferred_element_type=jnp.float32) + b_ref[...]
    o_ref[...] = jnp.maximum(z, 0.0)


def _tc1(p, x, dp, w1, b1):
    return pl.pallas_call(
        _tc1_body,
        grid=(N // BN,),
        in_specs=[
            pl.BlockSpec((NC, BN, D), lambda i: (0, i, 0)),
            pl.BlockSpec((BN, D), lambda i: (i, 0)),
            pl.BlockSpec((NC, BN, 1), lambda i: (0, i, 0)),
            pl.BlockSpec((D, D), lambda i: (0, 0)),
            pl.BlockSpec((1, D), lambda i: (0, 0)),
        ],
        out_specs=pl.BlockSpec((BN, D), lambda i: (i, 0)),
        out_shape=jax.ShapeDtypeStruct((N, D), jnp.float32),
    )(p, x, dp, w1, b1)


def _tc2_body(q_ref, h1_ref, dp_ref, w2_ref, b2_ref, wd1_ref, bd1_ref,
              wd2_ref, bd2_ref, o_ref):
    inv = 1.0 / (dp_ref[0, :, 0:1] + dp_ref[1, :, 0:1] + 1.0)
    hn = (q_ref[0] + q_ref[1] + h1_ref[...]) * inv
    h2 = jnp.dot(hn, w2_ref[...], preferred_element_type=jnp.float32) + b2_ref[...]
    t = jnp.maximum(
        jnp.dot(h2, wd1_ref[...], preferred_element_type=jnp.float32)
        + bd1_ref[...], 0.0)
    o_ref[...] = (jnp.dot(t, wd2_ref[...], preferred_element_type=jnp.float32)
                  + bd2_ref[...])


def _tc2(q, h1, dp, w2, b2, wd1, bd1, wd2, bd2):
    return pl.pallas_call(
        _tc2_body,
        grid=(N // BN,),
        in_specs=[
            pl.BlockSpec((NC, BN, D), lambda i: (0, i, 0)),
            pl.BlockSpec((BN, D), lambda i: (i, 0)),
            pl.BlockSpec((NC, BN, 1), lambda i: (0, i, 0)),
            pl.BlockSpec((D, D), lambda i: (0, 0)),
            pl.BlockSpec((1, D), lambda i: (0, 0)),
            pl.BlockSpec((D, D), lambda i: (0, 0)),
            pl.BlockSpec((1, D), lambda i: (0, 0)),
            pl.BlockSpec((D, 1), lambda i: (0, 0)),
            pl.BlockSpec((1, 1), lambda i: (0, 0)),
        ],
        out_specs=pl.BlockSpec((BN, 1), lambda i: (i, 0)),
        out_shape=jax.ShapeDtypeStruct((N, 1), jnp.float32),
    )(q, h1, dp, w2, b2, wd1, bd1, wd2, bd2)


def kernel(x, edge_index, W1, b1, W2, b2, Wd1, bd1, Wd2, bd2):
    src = edge_index[0]
    dst = edge_index[1]
    agg_p, deg_flat = _sc_agg_deg(x, src, dst)
    deg_p = deg_flat.reshape(NC, N, 1)
    h1 = _tc1(agg_p, x, deg_p, W1, b1.reshape(1, D))
    agg2_p = _sc_agg(h1, src, dst)
    out = _tc2(agg2_p, h1, deg_p, W2, b2.reshape(1, D),
               Wd1, bd1.reshape(1, D), Wd2, bd2.reshape(1, 1))
    return out
